# Initial kernel scaffold; baseline (speedup 1.0000x reference)
import jax
import jax.numpy as jnp
from jax.experimental import pallas as pl


def kernel(x, edge_index, edge_attr, batch, W0, b0, Wh1, bh1, Wh2, bh2, Wroot, bconv, gru_Wih, gru_Whh, gru_bih, gru_bhh, lstm_Wih, lstm_Whh, lstm_bih, lstm_bhh, W1, b1, W2, b2):
    raise NotImplementedError("write your pallas kernel here")


# speedup vs baseline: 3.3226x; 3.3226x over previous
import functools

import jax
import jax.numpy as jnp
from jax import lax
from jax.experimental import pallas as pl
from jax.experimental.pallas import tpu as pltpu
from jax.experimental.pallas import tpu_sc as plsc

F32 = jnp.float32

N = 10000
E = 160000
NUM_FEAT = 128
DIM = 16
B = 64

NPAD = 10240
EPAD = 163840
CH = 128
NW = 32
NCH = EPAD // (NW * CH)
EW = EPAD // NW
ROWS_PER_TILE = NPAD // 16


def _lin0_body(x_ref, w_ref, b_ref, o_ref):
    o_ref[...] = jax.nn.relu(
        jnp.dot(x_ref[...], w_ref[...], preferred_element_type=F32)
        + b_ref[...])


def _we_body(ea_ref, w1_ref, b1_ref, w2_ref, b2_ref, o_ref):
    h = jax.nn.relu(
        jnp.dot(ea_ref[...], w1_ref[...], preferred_element_type=F32)
        + b1_ref[...])
    o_ref[...] = jnp.dot(h, w2_ref[...], preferred_element_type=F32) + b2_ref[...]


def _msg_body(xj_ref, we_ref, r_ref, s_ref, o_ref):
    xjb = jnp.dot(xj_ref[...], r_ref[...], preferred_element_type=F32)
    prod = xjb * we_ref[...]
    o_ref[...] = jnp.dot(prod, s_ref[...], preferred_element_type=F32)


def _gru_body(h_ref, ag_ref, ct_ref, wroot_ref, bconv_ref,
              wir_ref, wiz_ref, win_ref, whr_ref, whz_ref, whn_ref,
              bir_ref, biz_ref, bin_ref, bhr_ref, bhz_ref, bhn_ref,
              o_ref):
    ag = ag_ref[0] + ag_ref[1]
    ct = jnp.maximum(ct_ref[0] + ct_ref[1], 1.0)
    aggr = ag / ct
    h = h_ref[...]
    m = jax.nn.relu(
        aggr + jnp.dot(h, wroot_ref[...], preferred_element_type=F32)
        + bconv_ref[...])
    gi_r = jnp.dot(m, wir_ref[...], preferred_element_type=F32) + bir_ref[...]
    gi_z = jnp.dot(m, wiz_ref[...], preferred_element_type=F32) + biz_ref[...]
    gi_n = jnp.dot(m, win_ref[...], preferred_element_type=F32) + bin_ref[...]
    gh_r = jnp.dot(h, whr_ref[...], preferred_element_type=F32) + bhr_ref[...]
    gh_z = jnp.dot(h, whz_ref[...], preferred_element_type=F32) + bhz_ref[...]
    gh_n = jnp.dot(h, whn_ref[...], preferred_element_type=F32) + bhn_ref[...]
    r = jax.nn.sigmoid(gi_r + gh_r)
    z = jax.nn.sigmoid(gi_z + gh_z)
    n = jnp.tanh(gi_n + r * gh_n)
    o_ref[...] = (1.0 - z) * n + z * h


def _s2s_body(st_ref, bc_ref, br_ref,
              li_i_ref, li_f_ref, li_g_ref, li_o_ref,
              lh_i_ref, lh_f_ref, lh_g_ref, lh_o_ref,
              lb_i_ref, lb_f_ref, lb_g_ref, lb_o_ref,
              w1_ref, b1_ref, w2_ref, b2_ref, o_ref):
    out = st_ref[...]
    bat_c = bc_ref[...]
    bat_r = br_ref[...]
    ids_r = lax.broadcasted_iota(jnp.int32, (1, B), 1)
    ids_c = lax.broadcasted_iota(jnp.int32, (B, 1), 0)
    M = (bat_c == ids_r).astype(F32)
    MT = (ids_c == bat_r).astype(F32)
    valid = (bat_c < B).astype(F32)

    q_star = jnp.zeros((B, 2 * DIM), F32)
    hl = jnp.zeros((B, DIM), F32)
    cl = jnp.zeros((B, DIM), F32)
    for _ in range(3):
        i_g = jax.nn.sigmoid(
            jnp.dot(q_star, li_i_ref[...], preferred_element_type=F32)
            + jnp.dot(hl, lh_i_ref[...], preferred_element_type=F32)
            + lb_i_ref[...])
        f_g = jax.nn.sigmoid(
            jnp.dot(q_star, li_f_ref[...], preferred_element_type=F32)
            + jnp.dot(hl, lh_f_ref[...], preferred_element_type=F32)
            + lb_f_ref[...])
        g_g = jnp.tanh(
            jnp.dot(q_star, li_g_ref[...], preferred_element_type=F32)
            + jnp.dot(hl, lh_g_ref[...], preferred_element_type=F32)
            + lb_g_ref[...])
        o_g = jax.nn.sigmoid(
            jnp.dot(q_star, li_o_ref[...], preferred_element_type=F32)
            + jnp.dot(hl, lh_o_ref[...], preferred_element_type=F32)
            + lb_o_ref[...])
        cl = f_g * cl + i_g * g_g
        hl = o_g * jnp.tanh(cl)
        q = hl
        qn = jnp.dot(M, q, preferred_element_type=F32)
        e = jnp.sum(out * qn, axis=1, keepdims=True)
        em = jnp.where(M > 0.0, e, -1e30)
        m_seg = jnp.max(em, axis=0, keepdims=True)
        m_seg = jnp.where(m_seg > -1e29, m_seg, 0.0)
        m_n = jnp.sum(M * m_seg, axis=1, keepdims=True)
        a = jnp.exp(e - m_n) * valid
        denom = jnp.sum(a * M, axis=0, keepdims=True)
        d_n = jnp.sum(M * denom, axis=1, keepdims=True)
        a = a / jnp.maximum(d_n, 1e-16)
        r_vec = jnp.dot(MT, a * out, preferred_element_type=F32)
        q_star = jnp.concatenate([q, r_vec], axis=1)
    hid = jax.nn.relu(
        jnp.dot(q_star, w1_ref[...], preferred_element_type=F32) + b1_ref[...])
    o_ref[...] = jnp.dot(hid, w2_ref[...], preferred_element_type=F32) + b2_ref[...]


def _sc_gather_body(table_hbm, src_hbm, xj_hbm, idx_v, rows_v, sem):
    c = lax.axis_index("c")
    s = lax.axis_index("s")
    wid = s * 2 + c
    base = wid * NCH
    pltpu.sync_copy(src_hbm.at[pl.ds(base, NCH)], idx_v)
    for w in range(NCH // 8):
        def fire(j, _, w=w):
            k = w * 8 + j
            pltpu.make_async_copy(
                table_hbm.at[idx_v.at[k]], rows_v.at[k], sem).start()
            return 0
        lax.fori_loop(0, 8, fire, 0)
        def drain(j, _, w=w):
            k = w * 8 + j
            pltpu.make_async_copy(
                table_hbm.at[idx_v.at[k]], rows_v.at[k], sem).wait()
            return 0
        lax.fori_loop(0, 8, drain, 0)
    pltpu.sync_copy(rows_v, xj_hbm.at[pl.ds(base, NCH)])


def _sc_scatter_body(val_hbm, dst_hbm, zeros_hbm, out_hbm, idx_v, val_v, acc_sh):
    c = lax.axis_index("c")
    s = lax.axis_index("s")
    wid = s * 2 + c
    base = wid * NCH
    rbase = s * ROWS_PER_TILE
    pltpu.sync_copy(zeros_hbm.at[pl.ds(rbase, ROWS_PER_TILE)],
                    acc_sh.at[pl.ds(rbase, ROWS_PER_TILE)])
    pltpu.sync_copy(dst_hbm.at[pl.ds(base, NCH)], idx_v)
    pltpu.sync_copy(val_hbm.at[pl.ds(base, NCH)], val_v)
    plsc.subcore_barrier()

    def body(j, _):
        pltpu.sync_copy(val_v.at[j], acc_sh.at[idx_v.at[j]], add=True)
        return 0
    lax.fori_loop(0, NCH, body, 0)
    plsc.subcore_barrier()
    pltpu.sync_copy(acc_sh.at[pl.ds(rbase, ROWS_PER_TILE)],
                    out_hbm.at[c].at[pl.ds(rbase, ROWS_PER_TILE)])


@functools.cache
def _sc_kernels():
    mesh = plsc.VectorSubcoreMesh(core_axis_name="c", subcore_axis_name="s")
    params = pltpu.CompilerParams(use_tc_tiling_on_sc=False)
    gather = pl.kernel(
        _sc_gather_body,
        out_type=jax.ShapeDtypeStruct((NW * NCH, CH, DIM), F32),
        mesh=mesh,
        compiler_params=params,
        scratch_types=[
            pltpu.VMEM((NCH, CH), jnp.int32),
            pltpu.VMEM((NCH, CH, DIM), F32),
            pltpu.SemaphoreType.DMA,
        ],
    )
    scatter = pl.kernel(
        _sc_scatter_body,
        out_type=jax.ShapeDtypeStruct((2, NPAD, DIM), F32),
        mesh=mesh,
        compiler_params=params,
        scratch_types=[
            pltpu.VMEM((NCH, CH), jnp.int32),
            pltpu.VMEM((NCH, CH, DIM), F32),
            pltpu.VMEM_SHARED((NPAD, DIM), F32),
        ],
    )
    return gather, scatter


def _sc_gather(table, srcp):
    return _sc_kernels()[0](table, srcp)


def _sc_scatter_add(val, dstp, zeros):
    return _sc_kernels()[1](val, dstp, zeros)


def _full(shape):
    return pl.BlockSpec(shape, lambda *_: tuple(0 for _ in shape))


def _lin0_call(xp, W0, b0):
    blk = 2048
    return pl.pallas_call(
        _lin0_body,
        grid=(NPAD // blk,),
        in_specs=[
            pl.BlockSpec((blk, NUM_FEAT), lambda i: (i, 0)),
            _full((NUM_FEAT, DIM)),
            _full((1, DIM)),
        ],
        out_specs=pl.BlockSpec((blk, DIM), lambda i: (i, 0)),
        out_shape=jax.ShapeDtypeStruct((NPAD, DIM), F32),
    )(xp, W0, b0)


def _we_call(eap, Wh1, bh1, Wh2, bh2):
    blk = 2048
    return pl.pallas_call(
        _we_body,
        grid=(EPAD // blk,),
        in_specs=[
            pl.BlockSpec((blk, 4), lambda i: (i, 0)),
            _full((4, 128)),
            _full((1, 128)),
            _full((128, DIM * DIM)),
            _full((1, DIM * DIM)),
        ],
        out_specs=pl.BlockSpec((blk, DIM * DIM), lambda i: (i, 0)),
        out_shape=jax.ShapeDtypeStruct((EPAD, DIM * DIM), F32),
    )(eap, Wh1, bh1, Wh2, bh2)


def _msg_call(xj, We2, R, S):
    blk = 2048
    return pl.pallas_call(
        _msg_body,
        grid=(EPAD // blk,),
        in_specs=[
            pl.BlockSpec((blk, DIM), lambda i: (i, 0)),
            pl.BlockSpec((blk, DIM * DIM), lambda i: (i, 0)),
            _full((DIM, DIM * DIM)),
            _full((DIM * DIM, DIM)),
        ],
        out_specs=pl.BlockSpec((blk, DIM), lambda i: (i, 0)),
        out_shape=jax.ShapeDtypeStruct((EPAD, DIM), F32),
    )(xj, We2, R, S)


def _gru_call(h, ag2, ct2, Wroot, bconv, wmats, bvecs):
    blk = 2048
    sm = _full((DIM, DIM))
    sb = _full((1, DIM))
    return pl.pallas_call(
        _gru_body,
        grid=(NPAD // blk,),
        in_specs=[
            pl.BlockSpec((blk, DIM), lambda i: (i, 0)),
            pl.BlockSpec((2, blk, DIM), lambda i: (0, i, 0)),
            pl.BlockSpec((2, blk, DIM), lambda i: (0, i, 0)),
            sm, sb,
            sm, sm, sm, sm, sm, sm,
            sb, sb, sb, sb, sb, sb,
        ],
        out_specs=pl.BlockSpec((blk, DIM), lambda i: (i, 0)),
        out_shape=jax.ShapeDtypeStruct((NPAD, DIM), F32),
    )(h, ag2, ct2, Wroot, bconv, *wmats, *bvecs)


def _s2s_call(h, bat_c, bat_r, lstm_args, W1, b1, W2, b2):
    in_specs = [
        _full((NPAD, DIM)),
        _full((NPAD, 1)),
        _full((1, NPAD)),
    ]
    for a in lstm_args:
        in_specs.append(_full(a.shape))
    in_specs += [_full((2 * DIM, DIM)), _full((1, DIM)),
                 _full((DIM, 1)), _full((1, 1))]
    return pl.pallas_call(
        _s2s_body,
        in_specs=in_specs,
        out_specs=_full((B, 1)),
        out_shape=jax.ShapeDtypeStruct((B, 1), F32),
    )(h, bat_c, bat_r, *lstm_args, W1, b1, W2, b2)


def kernel(x, edge_index, edge_attr, batch, W0, b0, Wh1, bh1, Wh2, bh2,
           Wroot, bconv, gru_Wih, gru_Whh, gru_bih, gru_bhh,
           lstm_Wih, lstm_Whh, lstm_bih, lstm_bhh, W1, b1, W2, b2):
    src = edge_index[0].astype(jnp.int32)
    dst = edge_index[1].astype(jnp.int32)

    xp = jnp.pad(x, ((0, NPAD - N), (0, 0)))
    eap = jnp.pad(edge_attr, ((0, EPAD - E), (0, 0)))
    srcp = jnp.pad(src, (0, EPAD - E)).reshape(NW * NCH, CH)
    dstp = jnp.pad(dst, (0, EPAD - E), constant_values=N).reshape(NW * NCH, CH)
    bat_c = jnp.pad(batch.astype(jnp.int32), (0, NPAD - N),
                    constant_values=127).reshape(NPAD, 1)
    bat_r = bat_c.reshape(1, NPAD)
    zerosN = jnp.zeros((NPAD, DIM), F32)
    onesE = jnp.ones((NW * NCH, CH, DIM), F32)

    R = (lax.broadcasted_iota(jnp.int32, (DIM, DIM * DIM), 1) // DIM
         == lax.broadcasted_iota(jnp.int32, (DIM, DIM * DIM), 0)).astype(F32)
    S = (lax.broadcasted_iota(jnp.int32, (DIM * DIM, DIM), 0) % DIM
         == lax.broadcasted_iota(jnp.int32, (DIM * DIM, DIM), 1)).astype(F32)

    WihT = gru_Wih.T
    WhhT = gru_Whh.T
    wmats = (WihT[:, :DIM], WihT[:, DIM:2 * DIM], WihT[:, 2 * DIM:],
             WhhT[:, :DIM], WhhT[:, DIM:2 * DIM], WhhT[:, 2 * DIM:])
    bvecs = (gru_bih[:DIM].reshape(1, DIM),
             gru_bih[DIM:2 * DIM].reshape(1, DIM),
             gru_bih[2 * DIM:].reshape(1, DIM),
             gru_bhh[:DIM].reshape(1, DIM),
             gru_bhh[DIM:2 * DIM].reshape(1, DIM),
             gru_bhh[2 * DIM:].reshape(1, DIM))

    LiT = lstm_Wih.T
    LhT = lstm_Whh.T
    lb = (lstm_bih + lstm_bhh)
    lstm_args = (LiT[:, :DIM], LiT[:, DIM:2 * DIM],
                 LiT[:, 2 * DIM:3 * DIM], LiT[:, 3 * DIM:],
                 LhT[:, :DIM], LhT[:, DIM:2 * DIM],
                 LhT[:, 2 * DIM:3 * DIM], LhT[:, 3 * DIM:],
                 lb[:DIM].reshape(1, DIM), lb[DIM:2 * DIM].reshape(1, DIM),
                 lb[2 * DIM:3 * DIM].reshape(1, DIM), lb[3 * DIM:].reshape(1, DIM))

    h = _lin0_call(xp, W0, b0.reshape(1, DIM))
    We2 = _we_call(eap, Wh1, bh1.reshape(1, 128), Wh2, bh2.reshape(1, DIM * DIM))
    cnt2 = _sc_scatter_add(onesE, dstp, zerosN)

    for _ in range(3):
        xj3 = _sc_gather(h, srcp)
        msg = _msg_call(xj3.reshape(EPAD, DIM), We2, R, S)
        ag2 = _sc_scatter_add(msg.reshape(NW * NCH, CH, DIM), dstp, zerosN)
        h = _gru_call(h, ag2, cnt2, Wroot, bconv.reshape(1, DIM), wmats, bvecs)

    y = _s2s_call(h, bat_c, bat_r, lstm_args,
                  W1, b1.reshape(1, DIM), W2, b2.reshape(1, 1))
    return y.reshape(-1)
